# baseline (speedup 1.0000x reference)
import functools

import jax
import jax.numpy as jnp
from jax import lax
from jax.experimental import pallas as pl
from jax.experimental.pallas import tpu as pltpu
from jax.experimental.pallas import tpu_sc as plsc

NPOINT = 512
RADIUS_LIST = [0.1, 0.2, 0.4]
NSAMPLE_LIST = [16, 32, 128]
B, N = 8, 2048
CPTS = 64
CTAB = 128
TM = 8192
GATHER_WIN = 128


def _fps_body(xt_ref, nx_ref):
    X = xt_ref[:, 0, :]
    Y = xt_ref[:, 1, :]
    Z = xt_ref[:, 2, :]
    n = X.shape[1]
    iota = lax.broadcasted_iota(jnp.int32, (1, n), 1)

    def body(i, carry):
        dist, far = carry
        oh = iota == far
        cx = jnp.sum(jnp.where(oh, X, 0.0), axis=1, keepdims=True)
        cy = jnp.sum(jnp.where(oh, Y, 0.0), axis=1, keepdims=True)
        cz = jnp.sum(jnp.where(oh, Z, 0.0), axis=1, keepdims=True)
        nx_ref[:, pl.ds(i, 1), :] = jnp.concatenate(
            [cx, cy, cz], axis=1)[:, None, :]
        dx = X - cx
        dy = Y - cy
        dz = Z - cz
        d = (dx * dx + dy * dy) + dz * dz
        dist = jnp.minimum(dist, d)
        maxv = jnp.max(dist, axis=1, keepdims=True)
        cand = jnp.where(dist == maxv, iota, n)
        far = jnp.min(cand, axis=1, keepdims=True)
        return dist, far

    dist0 = jnp.full((B, N), 1e10, dtype=jnp.float32)
    far0 = jnp.zeros((B, 1), dtype=jnp.int32)
    lax.fori_loop(0, NPOINT, body, (dist0, far0))


def _run_fps(xyz_t):
    return pl.pallas_call(
        _fps_body,
        out_shape=jax.ShapeDtypeStruct((B, NPOINT, 3), jnp.float32),
    )(xyz_t)


def _ballq_body(xt_ref, nx_ref, g16_ref, g32_ref, g128_ref, scr_ref):
    S = NPOINT
    P = xt_ref[0]
    C = nx_ref[0]
    b = pl.program_id(0)
    boff = b * N

    dotT = lax.dot_general(
        P.astype(jnp.bfloat16), C.astype(jnp.bfloat16),
        (((0,), (1,)), ((), ())), preferred_element_type=jnp.float32)
    c2 = (C[:, 0:1] * C[:, 0:1] + C[:, 1:2] * C[:, 1:2]) + C[:, 2:3] * C[:, 2:3]
    p2 = (P[0:1, :] * P[0:1, :] + P[1:2, :] * P[1:2, :]) + P[2:3, :] * P[2:3, :]
    sqr = ((-2.0 * dotT) + jnp.transpose(c2)) + jnp.transpose(p2)

    for r, K, gref in zip(RADIUS_LIST, NSAMPLE_LIST,
                          (g16_ref, g32_ref, g128_ref)):
        mask = (sqr <= jnp.float32(r * r)).astype(jnp.int32)
        c = mask
        sh = 1
        while sh < N:
            c = c + jnp.concatenate(
                [jnp.zeros((sh, S), jnp.int32), c[:-sh, :]], axis=0)
            sh *= 2

        cb = jnp.minimum(c, K + 1).astype(jnp.bfloat16)
        ones_row = jnp.ones((1, N), jnp.bfloat16)

        def kbody(k, _):
            kb = k.astype(jnp.bfloat16)
            cm = jnp.where(cb <= kb, jnp.bfloat16(1), jnp.bfloat16(0))
            cnt = lax.dot_general(
                ones_row, cm, (((1,), (0,)), ((), ())),
                preferred_element_type=jnp.float32).astype(jnp.int32)
            scr_ref[pl.ds(k, 1)] = cnt[None]
            return 0

        lax.fori_loop(0, K, kbody, 0)
        idxs = scr_ref[0:K, 0, :]
        first = idxs[0:1, :]
        gref[0] = jnp.minimum(jnp.where(idxs == N, first, idxs), N - 1) + boff


def _run_ballq(xyz_t, new_xyz):
    S = NPOINT
    return pl.pallas_call(
        _ballq_body,
        grid=(B,),
        in_specs=[
            pl.BlockSpec((1, 3, N), lambda i: (i, 0, 0)),
            pl.BlockSpec((1, S, 3), lambda i: (i, 0, 0)),
        ],
        out_specs=[
            pl.BlockSpec((1, K, S), lambda i: (i, 0, 0)) for K in NSAMPLE_LIST
        ],
        out_shape=[
            jax.ShapeDtypeStruct((B, K, S), jnp.int32) for K in NSAMPLE_LIST
        ],
        scratch_shapes=[pltpu.VMEM((max(NSAMPLE_LIST), 1, S), jnp.int32)],
        compiler_params=pltpu.CompilerParams(
            dimension_semantics=("arbitrary",)),
    )(xyz_t, new_xyz)


def _sc_gather(table, gidx_flat, m):
    mesh = plsc.VectorSubcoreMesh(core_axis_name="core",
                                  subcore_axis_name="subcore")

    @functools.partial(
        pl.kernel, mesh=mesh,
        out_type=jax.ShapeDtypeStruct((m, CTAB), jnp.float32))
    def k(tab_hbm, idx_hbm, out_hbm):
        def body(i_vmem, o_vmem):
            pltpu.sync_copy(tab_hbm.at[i_vmem.at[0]], o_vmem)

        pltpu.emit_pipeline(
            body,
            grid=(m // GATHER_WIN,),
            in_specs=[pl.BlockSpec((1, GATHER_WIN), index_map=lambda i: (0, i))],
            out_specs=[pl.BlockSpec((GATHER_WIN, CTAB),
                                    index_map=lambda i: (i, 0))],
            core_axis_name=("core", "subcore"),
            dimension_semantics=(pltpu.PARALLEL,),
        )(idx_hbm, out_hbm)

    return k(table, gidx_flat.reshape(1, m))


def _layer1_body(g_ref, nxz_ref, w_ref, b_ref, x_ref, st_ref, *, K):
    g = g_ref[...]
    tk = TM // K
    cent = nxz_ref[...]
    centk = jnp.broadcast_to(cent[:, None, :], (tk, K, 3)).reshape(TM, 3)
    gxyz = g[:, 64:67] - centk
    xin = jnp.concatenate(
        [g[:, :64].astype(jnp.bfloat16), gxyz.astype(jnp.bfloat16),
         jnp.zeros((TM, CTAB - 67), jnp.bfloat16)], axis=1)
    x = lax.dot_general(
        xin, w_ref[...].astype(jnp.bfloat16),
        (((1,), (0,)), ((), ())), preferred_element_type=jnp.float32)
    x = x + b_ref[...]
    x_ref[...] = x.astype(jnp.bfloat16)
    c = x.shape[1]
    part = jnp.concatenate(
        [jnp.sum(x, axis=0, keepdims=True),
         jnp.sum(x * x, axis=0, keepdims=True),
         jnp.zeros((6, c), jnp.float32)], axis=0)

    @pl.when(pl.program_id(0) == 0)
    def _():
        st_ref[...] = part

    @pl.when(pl.program_id(0) != 0)
    def _():
        st_ref[...] = st_ref[...] + part


def _layer_mid_body(x_ref, st_in_ref, w_ref, b_ref, gamma_ref, beta_ref,
                    y_ref, st_ref, *, m_count):
    x = x_ref[...].astype(jnp.float32)
    s = st_in_ref[0:1, :]
    ss = st_in_ref[1:2, :]
    mean = s / m_count
    var = ss / m_count - mean * mean
    a = gamma_ref[...] / jnp.sqrt(var + 1e-5)
    cshift = beta_ref[...] - a * mean
    h = jnp.maximum(x * a + cshift, 0.0)
    y = lax.dot_general(
        h.astype(jnp.bfloat16), w_ref[...].astype(jnp.bfloat16),
        (((1,), (0,)), ((), ())), preferred_element_type=jnp.float32)
    y = y + b_ref[...]
    y_ref[...] = y.astype(jnp.bfloat16)
    c = y.shape[1]
    part = jnp.concatenate(
        [jnp.sum(y, axis=0, keepdims=True),
         jnp.sum(y * y, axis=0, keepdims=True),
         jnp.zeros((6, c), jnp.float32)], axis=0)

    @pl.when(pl.program_id(0) == 0)
    def _():
        st_ref[...] = part

    @pl.when(pl.program_id(0) != 0)
    def _():
        st_ref[...] = st_ref[...] + part


def _final_body(x_ref, st_in_ref, gamma_ref, beta_ref, o_ref, *, m_count, K):
    x = x_ref[...].astype(jnp.float32)
    s = st_in_ref[0:1, :]
    ss = st_in_ref[1:2, :]
    mean = s / m_count
    var = ss / m_count - mean * mean
    a = gamma_ref[...] / jnp.sqrt(var + 1e-5)
    cshift = beta_ref[...] - a * mean
    h = jnp.maximum(x * a + cshift, 0.0)
    c = x.shape[1]
    o_ref[...] = jnp.max(h.reshape(TM // K, K, c), axis=1)[None]


def _run_mlp(g, nxz_flat, params, K, m):
    steps = m // TM
    acc = pltpu.CompilerParams(dimension_semantics=("arbitrary",))
    full = lambda shape: pl.BlockSpec(shape, lambda i: tuple(0 for _ in shape))

    (w1, b1, g1, be1), (w2, b2, g2, be2), (w3, b3, g3, be3) = params
    c1, c2r, c3 = w1.shape[0], w2.shape[0], w3.shape[0]
    w1p = jnp.zeros((CTAB, c1), jnp.float32).at[:67, :].set(w1.T)
    w2p, w3p = w2.T, w3.T

    x1, st1 = pl.pallas_call(
        functools.partial(_layer1_body, K=K),
        grid=(steps,),
        in_specs=[
            pl.BlockSpec((TM, CTAB), lambda i: (i, 0)),
            pl.BlockSpec((TM // K, 3), lambda i: (i, 0)),
            full((CTAB, c1)),
            full((1, c1)),
        ],
        out_specs=[pl.BlockSpec((TM, c1), lambda i: (i, 0)), full((8, c1))],
        out_shape=[jax.ShapeDtypeStruct((m, c1), jnp.bfloat16),
                   jax.ShapeDtypeStruct((8, c1), jnp.float32)],
        compiler_params=acc,
    )(g, nxz_flat, w1p, b1[None, :])

    x2, st2 = pl.pallas_call(
        functools.partial(_layer_mid_body, m_count=float(m)),
        grid=(steps,),
        in_specs=[
            pl.BlockSpec((TM, c1), lambda i: (i, 0)),
            full((8, c1)),
            full((c1, c2r)),
            full((1, c2r)),
            full((1, c1)),
            full((1, c1)),
        ],
        out_specs=[pl.BlockSpec((TM, c2r), lambda i: (i, 0)), full((8, c2r))],
        out_shape=[jax.ShapeDtypeStruct((m, c2r), jnp.bfloat16),
                   jax.ShapeDtypeStruct((8, c2r), jnp.float32)],
        compiler_params=acc,
    )(x1, st1, w2p, b2[None, :], g1[None, :], be1[None, :])

    x3, st3 = pl.pallas_call(
        functools.partial(_layer_mid_body, m_count=float(m)),
        grid=(steps,),
        in_specs=[
            pl.BlockSpec((TM, c2r), lambda i: (i, 0)),
            full((8, c2r)),
            full((c2r, c3)),
            full((1, c3)),
            full((1, c2r)),
            full((1, c2r)),
        ],
        out_specs=[pl.BlockSpec((TM, c3), lambda i: (i, 0)), full((8, c3))],
        out_shape=[jax.ShapeDtypeStruct((m, c3), jnp.bfloat16),
                   jax.ShapeDtypeStruct((8, c3), jnp.float32)],
        compiler_params=acc,
    )(x2, st2, w3p, b3[None, :], g2[None, :], be2[None, :])

    out = pl.pallas_call(
        functools.partial(_final_body, m_count=float(m), K=K),
        grid=(steps,),
        in_specs=[
            pl.BlockSpec((TM, c3), lambda i: (i, 0)),
            full((8, c3)),
            full((1, c3)),
            full((1, c3)),
        ],
        out_specs=pl.BlockSpec((1, TM // K, c3), lambda i: (i, 0, 0)),
        out_shape=jax.ShapeDtypeStruct((steps, TM // K, c3), jnp.float32),
        compiler_params=acc,
    )(x3, st3, g3[None, :], be3[None, :])
    return out.reshape(B * NPOINT, c3)


def kernel(xyz, points, params):
    S = NPOINT
    xyz_t = jnp.transpose(xyz, (0, 2, 1))
    new_xyz = _run_fps(xyz_t)
    gidx = _run_ballq(xyz_t, new_xyz)

    table = jnp.concatenate(
        [points, xyz, jnp.zeros((B, N, CTAB - 67), jnp.float32)],
        axis=-1).reshape(B * N, CTAB)
    nxz_flat = new_xyz.reshape(B * S, 3)

    outs = []
    for i, K in enumerate(NSAMPLE_LIST):
        m = B * S * K
        gi = jnp.transpose(gidx[i], (0, 2, 1)).reshape(-1)
        g = _sc_gather(table, gi, m)
        outs.append(_run_mlp(g, nxz_flat, params[i], K, m))

    new_points = jnp.concatenate(outs, axis=-1).reshape(B, S, -1)
    return (new_xyz, new_points)


# speedup vs baseline: 1.0807x; 1.0807x over previous
import functools

import jax
import jax.numpy as jnp
from jax import lax
from jax.experimental import pallas as pl
from jax.experimental.pallas import tpu as pltpu
from jax.experimental.pallas import tpu_sc as plsc

NPOINT = 512
RADIUS_LIST = [0.1, 0.2, 0.4]
NSAMPLE_LIST = [16, 32, 128]
B, N = 8, 2048
CPTS = 64
CTAB = 128
TM = 8192
GATHER_WIN = 128


def _fps_body(xt_ref, nx_ref):
    X = xt_ref[:, 0, :]
    Y = xt_ref[:, 1, :]
    Z = xt_ref[:, 2, :]
    n = X.shape[1]
    iota = lax.broadcasted_iota(jnp.int32, (1, n), 1)

    def body(i, carry):
        dist, far = carry
        oh = iota == far
        cx = jnp.sum(jnp.where(oh, X, 0.0), axis=1, keepdims=True)
        cy = jnp.sum(jnp.where(oh, Y, 0.0), axis=1, keepdims=True)
        cz = jnp.sum(jnp.where(oh, Z, 0.0), axis=1, keepdims=True)
        nx_ref[:, pl.ds(i, 1), :] = jnp.concatenate(
            [cx, cy, cz], axis=1)[:, None, :]
        dx = X - cx
        dy = Y - cy
        dz = Z - cz
        d = (dx * dx + dy * dy) + dz * dz
        dist = jnp.minimum(dist, d)
        maxv = jnp.max(dist, axis=1, keepdims=True)
        cand = jnp.where(dist == maxv, iota, n)
        far = jnp.min(cand, axis=1, keepdims=True)
        return dist, far

    dist0 = jnp.full((B, N), 1e10, dtype=jnp.float32)
    far0 = jnp.zeros((B, 1), dtype=jnp.int32)
    lax.fori_loop(0, NPOINT, body, (dist0, far0))


def _run_fps(xyz_t):
    return pl.pallas_call(
        _fps_body,
        out_shape=jax.ShapeDtypeStruct((B, NPOINT, 3), jnp.float32),
    )(xyz_t)


def _ballq_body(xt_ref, nx_ref, g16_ref, g32_ref, g128_ref, scr_ref):
    S = NPOINT
    P = xt_ref[0]
    C = nx_ref[0]
    b = pl.program_id(0)
    boff = b * N

    dotT = lax.dot_general(
        P.astype(jnp.bfloat16), C.astype(jnp.bfloat16),
        (((0,), (1,)), ((), ())), preferred_element_type=jnp.float32)
    c2 = (C[:, 0:1] * C[:, 0:1] + C[:, 1:2] * C[:, 1:2]) + C[:, 2:3] * C[:, 2:3]
    p2 = (P[0:1, :] * P[0:1, :] + P[1:2, :] * P[1:2, :]) + P[2:3, :] * P[2:3, :]
    sqr = ((-2.0 * dotT) + jnp.transpose(c2)) + jnp.transpose(p2)

    for r, K, gref in zip(RADIUS_LIST, NSAMPLE_LIST,
                          (g16_ref, g32_ref, g128_ref)):
        mask = (sqr <= jnp.float32(r * r)).astype(jnp.int32)
        c = mask
        sh = 1
        while sh < N:
            c = c + jnp.concatenate(
                [jnp.zeros((sh, S), jnp.int32), c[:-sh, :]], axis=0)
            sh *= 2

        cb = jnp.minimum(c, K + 1).astype(jnp.bfloat16)
        ones_row = jnp.ones((1, N), jnp.bfloat16)

        def kbody(j, _):
            base = j * 8
            for t in range(8):
                kb = (base + t).astype(jnp.bfloat16)
                cm = jnp.where(cb <= kb, jnp.bfloat16(1), jnp.bfloat16(0))
                cnt = lax.dot_general(
                    ones_row, cm, (((1,), (0,)), ((), ())),
                    preferred_element_type=jnp.float32).astype(jnp.int32)
                scr_ref[pl.ds(base + t, 1)] = cnt[None]
            return 0

        lax.fori_loop(0, K // 8, kbody, 0)
        idxs = scr_ref[0:K, 0, :]
        first = idxs[0:1, :]
        gref[0] = jnp.minimum(jnp.where(idxs == N, first, idxs), N - 1) + boff


def _run_ballq(xyz_t, new_xyz):
    S = NPOINT
    return pl.pallas_call(
        _ballq_body,
        grid=(B,),
        in_specs=[
            pl.BlockSpec((1, 3, N), lambda i: (i, 0, 0)),
            pl.BlockSpec((1, S, 3), lambda i: (i, 0, 0)),
        ],
        out_specs=[
            pl.BlockSpec((1, K, S), lambda i: (i, 0, 0)) for K in NSAMPLE_LIST
        ],
        out_shape=[
            jax.ShapeDtypeStruct((B, K, S), jnp.int32) for K in NSAMPLE_LIST
        ],
        scratch_shapes=[pltpu.VMEM((max(NSAMPLE_LIST), 1, S), jnp.int32)],
        compiler_params=pltpu.CompilerParams(
            dimension_semantics=("arbitrary",)),
    )(xyz_t, new_xyz)


def _sc_gather(table, gidx_flat, m):
    mesh = plsc.VectorSubcoreMesh(core_axis_name="core",
                                  subcore_axis_name="subcore")

    @functools.partial(
        pl.kernel, mesh=mesh,
        out_type=jax.ShapeDtypeStruct((m, CTAB), jnp.float32))
    def k(tab_hbm, idx_hbm, out_hbm):
        def body(i_vmem, o_vmem):
            pltpu.sync_copy(tab_hbm.at[i_vmem.at[0]], o_vmem)

        pltpu.emit_pipeline(
            body,
            grid=(m // GATHER_WIN,),
            in_specs=[pl.BlockSpec((1, GATHER_WIN), index_map=lambda i: (0, i))],
            out_specs=[pl.BlockSpec((GATHER_WIN, CTAB),
                                    index_map=lambda i: (i, 0))],
            core_axis_name=("core", "subcore"),
            dimension_semantics=(pltpu.PARALLEL,),
        )(idx_hbm, out_hbm)

    return k(table, gidx_flat.reshape(1, m))


def _layer1_body(g_ref, nxz_ref, w_ref, b_ref, x_ref, st_ref, *, K):
    g = g_ref[...]
    tk = TM // K
    cent = nxz_ref[...]
    centk = jnp.broadcast_to(cent[:, None, :], (tk, K, 3)).reshape(TM, 3)
    gxyz = g[:, 64:67] - centk
    xin = jnp.concatenate(
        [g[:, :64].astype(jnp.bfloat16), gxyz.astype(jnp.bfloat16),
         jnp.zeros((TM, CTAB - 67), jnp.bfloat16)], axis=1)
    x = lax.dot_general(
        xin, w_ref[...].astype(jnp.bfloat16),
        (((1,), (0,)), ((), ())), preferred_element_type=jnp.float32)
    x = x + b_ref[...]
    x_ref[...] = x.astype(jnp.bfloat16)
    c = x.shape[1]
    part = jnp.concatenate(
        [jnp.sum(x, axis=0, keepdims=True),
         jnp.sum(x * x, axis=0, keepdims=True),
         jnp.zeros((6, c), jnp.float32)], axis=0)

    @pl.when(pl.program_id(0) == 0)
    def _():
        st_ref[...] = part

    @pl.when(pl.program_id(0) != 0)
    def _():
        st_ref[...] = st_ref[...] + part


def _layer_mid_body(x_ref, st_in_ref, w_ref, b_ref, gamma_ref, beta_ref,
                    y_ref, st_ref, *, m_count):
    x = x_ref[...].astype(jnp.float32)
    s = st_in_ref[0:1, :]
    ss = st_in_ref[1:2, :]
    mean = s / m_count
    var = ss / m_count - mean * mean
    a = gamma_ref[...] / jnp.sqrt(var + 1e-5)
    cshift = beta_ref[...] - a * mean
    h = jnp.maximum(x * a + cshift, 0.0)
    y = lax.dot_general(
        h.astype(jnp.bfloat16), w_ref[...].astype(jnp.bfloat16),
        (((1,), (0,)), ((), ())), preferred_element_type=jnp.float32)
    y = y + b_ref[...]
    y_ref[...] = y.astype(jnp.bfloat16)
    c = y.shape[1]
    part = jnp.concatenate(
        [jnp.sum(y, axis=0, keepdims=True),
         jnp.sum(y * y, axis=0, keepdims=True),
         jnp.zeros((6, c), jnp.float32)], axis=0)

    @pl.when(pl.program_id(0) == 0)
    def _():
        st_ref[...] = part

    @pl.when(pl.program_id(0) != 0)
    def _():
        st_ref[...] = st_ref[...] + part


def _final_body(x_ref, st_in_ref, gamma_ref, beta_ref, o_ref, *, m_count, K):
    x = x_ref[...].astype(jnp.float32)
    s = st_in_ref[0:1, :]
    ss = st_in_ref[1:2, :]
    mean = s / m_count
    var = ss / m_count - mean * mean
    a = gamma_ref[...] / jnp.sqrt(var + 1e-5)
    cshift = beta_ref[...] - a * mean
    h = jnp.maximum(x * a + cshift, 0.0)
    c = x.shape[1]
    o_ref[...] = jnp.max(h.reshape(TM // K, K, c), axis=1)[None]


def _run_mlp(g, nxz_flat, params, K, m):
    steps = m // TM
    acc = pltpu.CompilerParams(dimension_semantics=("arbitrary",))
    full = lambda shape: pl.BlockSpec(shape, lambda i: tuple(0 for _ in shape))

    (w1, b1, g1, be1), (w2, b2, g2, be2), (w3, b3, g3, be3) = params
    c1, c2r, c3 = w1.shape[0], w2.shape[0], w3.shape[0]
    w1p = jnp.zeros((CTAB, c1), jnp.float32).at[:67, :].set(w1.T)
    w2p, w3p = w2.T, w3.T

    x1, st1 = pl.pallas_call(
        functools.partial(_layer1_body, K=K),
        grid=(steps,),
        in_specs=[
            pl.BlockSpec((TM, CTAB), lambda i: (i, 0)),
            pl.BlockSpec((TM // K, 3), lambda i: (i, 0)),
            full((CTAB, c1)),
            full((1, c1)),
        ],
        out_specs=[pl.BlockSpec((TM, c1), lambda i: (i, 0)), full((8, c1))],
        out_shape=[jax.ShapeDtypeStruct((m, c1), jnp.bfloat16),
                   jax.ShapeDtypeStruct((8, c1), jnp.float32)],
        compiler_params=acc,
    )(g, nxz_flat, w1p, b1[None, :])

    x2, st2 = pl.pallas_call(
        functools.partial(_layer_mid_body, m_count=float(m)),
        grid=(steps,),
        in_specs=[
            pl.BlockSpec((TM, c1), lambda i: (i, 0)),
            full((8, c1)),
            full((c1, c2r)),
            full((1, c2r)),
            full((1, c1)),
            full((1, c1)),
        ],
        out_specs=[pl.BlockSpec((TM, c2r), lambda i: (i, 0)), full((8, c2r))],
        out_shape=[jax.ShapeDtypeStruct((m, c2r), jnp.bfloat16),
                   jax.ShapeDtypeStruct((8, c2r), jnp.float32)],
        compiler_params=acc,
    )(x1, st1, w2p, b2[None, :], g1[None, :], be1[None, :])

    x3, st3 = pl.pallas_call(
        functools.partial(_layer_mid_body, m_count=float(m)),
        grid=(steps,),
        in_specs=[
            pl.BlockSpec((TM, c2r), lambda i: (i, 0)),
            full((8, c2r)),
            full((c2r, c3)),
            full((1, c3)),
            full((1, c2r)),
            full((1, c2r)),
        ],
        out_specs=[pl.BlockSpec((TM, c3), lambda i: (i, 0)), full((8, c3))],
        out_shape=[jax.ShapeDtypeStruct((m, c3), jnp.bfloat16),
                   jax.ShapeDtypeStruct((8, c3), jnp.float32)],
        compiler_params=acc,
    )(x2, st2, w3p, b3[None, :], g2[None, :], be2[None, :])

    out = pl.pallas_call(
        functools.partial(_final_body, m_count=float(m), K=K),
        grid=(steps,),
        in_specs=[
            pl.BlockSpec((TM, c3), lambda i: (i, 0)),
            full((8, c3)),
            full((1, c3)),
            full((1, c3)),
        ],
        out_specs=pl.BlockSpec((1, TM // K, c3), lambda i: (i, 0, 0)),
        out_shape=jax.ShapeDtypeStruct((steps, TM // K, c3), jnp.float32),
        compiler_params=acc,
    )(x3, st3, g3[None, :], be3[None, :])
    return out.reshape(B * NPOINT, c3)


def kernel(xyz, points, params):
    S = NPOINT
    xyz_t = jnp.transpose(xyz, (0, 2, 1))
    new_xyz = _run_fps(xyz_t)
    gidx = _run_ballq(xyz_t, new_xyz)

    table = jnp.concatenate(
        [points, xyz, jnp.zeros((B, N, CTAB - 67), jnp.float32)],
        axis=-1).reshape(B * N, CTAB)
    nxz_flat = new_xyz.reshape(B * S, 3)

    outs = []
    for i, K in enumerate(NSAMPLE_LIST):
        m = B * S * K
        gi = jnp.transpose(gidx[i], (0, 2, 1)).reshape(-1)
        g = _sc_gather(table, gi, m)
        outs.append(_run_mlp(g, nxz_flat, params[i], K, m))

    new_points = jnp.concatenate(outs, axis=-1).reshape(B, S, -1)
    return (new_xyz, new_points)


# speedup vs baseline: 1.0824x; 1.0016x over previous
import functools

import jax
import jax.numpy as jnp
from jax import lax
from jax.experimental import pallas as pl
from jax.experimental.pallas import tpu as pltpu
from jax.experimental.pallas import tpu_sc as plsc

NPOINT = 512
RADIUS_LIST = [0.1, 0.2, 0.4]
NSAMPLE_LIST = [16, 32, 128]
B, N = 8, 2048
CPTS = 64
CTAB = 128
TM = 8192
GATHER_WIN = 128


def _fps_body(xt_ref, nx_ref):
    X = xt_ref[:, 0, :]
    Y = xt_ref[:, 1, :]
    Z = xt_ref[:, 2, :]
    n = X.shape[1]
    iota = lax.broadcasted_iota(jnp.int32, (1, n), 1)

    def body(i, carry):
        dist, far = carry
        oh = iota == far
        cx = jnp.sum(jnp.where(oh, X, 0.0), axis=1, keepdims=True)
        cy = jnp.sum(jnp.where(oh, Y, 0.0), axis=1, keepdims=True)
        cz = jnp.sum(jnp.where(oh, Z, 0.0), axis=1, keepdims=True)
        nx_ref[:, pl.ds(i, 1), :] = jnp.concatenate(
            [cx, cy, cz], axis=1)[:, None, :]
        dx = X - cx
        dy = Y - cy
        dz = Z - cz
        d = (dx * dx + dy * dy) + dz * dz
        dist = jnp.minimum(dist, d)
        maxv = jnp.max(dist, axis=1, keepdims=True)
        cand = jnp.where(dist == maxv, iota, n)
        far = jnp.min(cand, axis=1, keepdims=True)
        return dist, far

    dist0 = jnp.full((B, N), 1e10, dtype=jnp.float32)
    far0 = jnp.zeros((B, 1), dtype=jnp.int32)
    lax.fori_loop(0, NPOINT, body, (dist0, far0))


def _run_fps(xyz_t):
    return pl.pallas_call(
        _fps_body,
        out_shape=jax.ShapeDtypeStruct((B, NPOINT, 3), jnp.float32),
    )(xyz_t)


def _ballq_body(xt_ref, nx_ref, g16_ref, g32_ref, g128_ref, scr_ref):
    S = NPOINT
    P = xt_ref[0]
    C = nx_ref[0]
    b = pl.program_id(0)
    boff = b * N

    dotT = lax.dot_general(
        P.astype(jnp.bfloat16), C.astype(jnp.bfloat16),
        (((0,), (1,)), ((), ())), preferred_element_type=jnp.float32)
    c2 = (C[:, 0:1] * C[:, 0:1] + C[:, 1:2] * C[:, 1:2]) + C[:, 2:3] * C[:, 2:3]
    p2 = (P[0:1, :] * P[0:1, :] + P[1:2, :] * P[1:2, :]) + P[2:3, :] * P[2:3, :]
    sqr = ((-2.0 * dotT) + jnp.transpose(c2)) + jnp.transpose(p2)

    for r, K, gref in zip(RADIUS_LIST, NSAMPLE_LIST,
                          (g16_ref, g32_ref, g128_ref)):
        mask = (sqr <= jnp.float32(r * r)).astype(jnp.int32)
        c = mask
        sh = 1
        while sh < N:
            c = c + jnp.concatenate(
                [jnp.zeros((sh, S), jnp.int32), c[:-sh, :]], axis=0)
            sh *= 2

        cb = jnp.minimum(c, K + 1).astype(jnp.bfloat16)
        ones_row = jnp.ones((1, N), jnp.bfloat16)

        def kbody(j, _):
            base = j * 8
            for t in range(8):
                kb = (base + t).astype(jnp.bfloat16)
                cm = jnp.where(cb <= kb, jnp.bfloat16(1), jnp.bfloat16(0))
                cnt = lax.dot_general(
                    ones_row, cm, (((1,), (0,)), ((), ())),
                    preferred_element_type=jnp.float32).astype(jnp.int32)
                scr_ref[pl.ds(base + t, 1)] = cnt[None]
            return 0

        lax.fori_loop(0, K // 8, kbody, 0)
        idxs = scr_ref[0:K, 0, :]
        first = idxs[0:1, :]
        gref[0] = jnp.minimum(jnp.where(idxs == N, first, idxs), N - 1) + boff


def _run_ballq(xyz_t, new_xyz):
    S = NPOINT
    return pl.pallas_call(
        _ballq_body,
        grid=(B,),
        in_specs=[
            pl.BlockSpec((1, 3, N), lambda i: (i, 0, 0)),
            pl.BlockSpec((1, S, 3), lambda i: (i, 0, 0)),
        ],
        out_specs=[
            pl.BlockSpec((1, K, S), lambda i: (i, 0, 0)) for K in NSAMPLE_LIST
        ],
        out_shape=[
            jax.ShapeDtypeStruct((B, K, S), jnp.int32) for K in NSAMPLE_LIST
        ],
        scratch_shapes=[pltpu.VMEM((max(NSAMPLE_LIST), 1, S), jnp.int32)],
        compiler_params=pltpu.CompilerParams(
            dimension_semantics=("parallel",)),
    )(xyz_t, new_xyz)


def _sc_gather(table, gidx_flat, m):
    mesh = plsc.VectorSubcoreMesh(core_axis_name="core",
                                  subcore_axis_name="subcore")

    @functools.partial(
        pl.kernel, mesh=mesh,
        out_type=jax.ShapeDtypeStruct((m, CTAB), jnp.float32))
    def k(tab_hbm, idx_hbm, out_hbm):
        def body(i_vmem, o_vmem):
            pltpu.sync_copy(tab_hbm.at[i_vmem.at[0]], o_vmem)

        pltpu.emit_pipeline(
            body,
            grid=(m // GATHER_WIN,),
            in_specs=[pl.BlockSpec((1, GATHER_WIN), index_map=lambda i: (0, i))],
            out_specs=[pl.BlockSpec((GATHER_WIN, CTAB),
                                    index_map=lambda i: (i, 0))],
            core_axis_name=("core", "subcore"),
            dimension_semantics=(pltpu.PARALLEL,),
        )(idx_hbm, out_hbm)

    return k(table, gidx_flat.reshape(1, m))


def _layer1_body(g_ref, nxz_ref, w_ref, b_ref, x_ref, st_ref, *, K):
    g = g_ref[...]
    tk = TM // K
    cent = nxz_ref[...]
    centk = jnp.broadcast_to(cent[:, None, :], (tk, K, 3)).reshape(TM, 3)
    gxyz = g[:, 64:67] - centk
    xin = jnp.concatenate(
        [g[:, :64].astype(jnp.bfloat16), gxyz.astype(jnp.bfloat16),
         jnp.zeros((TM, CTAB - 67), jnp.bfloat16)], axis=1)
    x = lax.dot_general(
        xin, w_ref[...].astype(jnp.bfloat16),
        (((1,), (0,)), ((), ())), preferred_element_type=jnp.float32)
    x = x + b_ref[...]
    x_ref[...] = x.astype(jnp.bfloat16)
    c = x.shape[1]
    part = jnp.concatenate(
        [jnp.sum(x, axis=0, keepdims=True),
         jnp.sum(x * x, axis=0, keepdims=True),
         jnp.zeros((6, c), jnp.float32)], axis=0)

    @pl.when(pl.program_id(0) == 0)
    def _():
        st_ref[...] = part

    @pl.when(pl.program_id(0) != 0)
    def _():
        st_ref[...] = st_ref[...] + part


def _layer_mid_body(x_ref, st_in_ref, w_ref, b_ref, gamma_ref, beta_ref,
                    y_ref, st_ref, *, m_count):
    x = x_ref[...].astype(jnp.float32)
    s = st_in_ref[0:1, :]
    ss = st_in_ref[1:2, :]
    mean = s / m_count
    var = ss / m_count - mean * mean
    a = gamma_ref[...] / jnp.sqrt(var + 1e-5)
    cshift = beta_ref[...] - a * mean
    h = jnp.maximum(x * a + cshift, 0.0)
    y = lax.dot_general(
        h.astype(jnp.bfloat16), w_ref[...].astype(jnp.bfloat16),
        (((1,), (0,)), ((), ())), preferred_element_type=jnp.float32)
    y = y + b_ref[...]
    y_ref[...] = y.astype(jnp.bfloat16)
    c = y.shape[1]
    part = jnp.concatenate(
        [jnp.sum(y, axis=0, keepdims=True),
         jnp.sum(y * y, axis=0, keepdims=True),
         jnp.zeros((6, c), jnp.float32)], axis=0)

    @pl.when(pl.program_id(0) == 0)
    def _():
        st_ref[...] = part

    @pl.when(pl.program_id(0) != 0)
    def _():
        st_ref[...] = st_ref[...] + part


def _final_body(x_ref, st_in_ref, gamma_ref, beta_ref, o_ref, *, m_count, K):
    x = x_ref[...].astype(jnp.float32)
    s = st_in_ref[0:1, :]
    ss = st_in_ref[1:2, :]
    mean = s / m_count
    var = ss / m_count - mean * mean
    a = gamma_ref[...] / jnp.sqrt(var + 1e-5)
    cshift = beta_ref[...] - a * mean
    h = jnp.maximum(x * a + cshift, 0.0)
    c = x.shape[1]
    o_ref[...] = jnp.max(h.reshape(TM // K, K, c), axis=1)[None]


def _run_mlp(g, nxz_flat, params, K, m):
    steps = m // TM
    acc = pltpu.CompilerParams(dimension_semantics=("arbitrary",))
    full = lambda shape: pl.BlockSpec(shape, lambda i: tuple(0 for _ in shape))

    (w1, b1, g1, be1), (w2, b2, g2, be2), (w3, b3, g3, be3) = params
    c1, c2r, c3 = w1.shape[0], w2.shape[0], w3.shape[0]
    w1p = jnp.zeros((CTAB, c1), jnp.float32).at[:67, :].set(w1.T)
    w2p, w3p = w2.T, w3.T

    x1, st1 = pl.pallas_call(
        functools.partial(_layer1_body, K=K),
        grid=(steps,),
        in_specs=[
            pl.BlockSpec((TM, CTAB), lambda i: (i, 0)),
            pl.BlockSpec((TM // K, 3), lambda i: (i, 0)),
            full((CTAB, c1)),
            full((1, c1)),
        ],
        out_specs=[pl.BlockSpec((TM, c1), lambda i: (i, 0)), full((8, c1))],
        out_shape=[jax.ShapeDtypeStruct((m, c1), jnp.bfloat16),
                   jax.ShapeDtypeStruct((8, c1), jnp.float32)],
        compiler_params=acc,
    )(g, nxz_flat, w1p, b1[None, :])

    x2, st2 = pl.pallas_call(
        functools.partial(_layer_mid_body, m_count=float(m)),
        grid=(steps,),
        in_specs=[
            pl.BlockSpec((TM, c1), lambda i: (i, 0)),
            full((8, c1)),
            full((c1, c2r)),
            full((1, c2r)),
            full((1, c1)),
            full((1, c1)),
        ],
        out_specs=[pl.BlockSpec((TM, c2r), lambda i: (i, 0)), full((8, c2r))],
        out_shape=[jax.ShapeDtypeStruct((m, c2r), jnp.bfloat16),
                   jax.ShapeDtypeStruct((8, c2r), jnp.float32)],
        compiler_params=acc,
    )(x1, st1, w2p, b2[None, :], g1[None, :], be1[None, :])

    x3, st3 = pl.pallas_call(
        functools.partial(_layer_mid_body, m_count=float(m)),
        grid=(steps,),
        in_specs=[
            pl.BlockSpec((TM, c2r), lambda i: (i, 0)),
            full((8, c2r)),
            full((c2r, c3)),
            full((1, c3)),
            full((1, c2r)),
            full((1, c2r)),
        ],
        out_specs=[pl.BlockSpec((TM, c3), lambda i: (i, 0)), full((8, c3))],
        out_shape=[jax.ShapeDtypeStruct((m, c3), jnp.bfloat16),
                   jax.ShapeDtypeStruct((8, c3), jnp.float32)],
        compiler_params=acc,
    )(x2, st2, w3p, b3[None, :], g2[None, :], be2[None, :])

    out = pl.pallas_call(
        functools.partial(_final_body, m_count=float(m), K=K),
        grid=(steps,),
        in_specs=[
            pl.BlockSpec((TM, c3), lambda i: (i, 0)),
            full((8, c3)),
            full((1, c3)),
            full((1, c3)),
        ],
        out_specs=pl.BlockSpec((1, TM // K, c3), lambda i: (i, 0, 0)),
        out_shape=jax.ShapeDtypeStruct((steps, TM // K, c3), jnp.float32),
        compiler_params=acc,
    )(x3, st3, g3[None, :], be3[None, :])
    return out.reshape(B * NPOINT, c3)


def kernel(xyz, points, params):
    S = NPOINT
    xyz_t = jnp.transpose(xyz, (0, 2, 1))
    new_xyz = _run_fps(xyz_t)
    gidx = _run_ballq(xyz_t, new_xyz)

    table = jnp.concatenate(
        [points, xyz, jnp.zeros((B, N, CTAB - 67), jnp.float32)],
        axis=-1).reshape(B * N, CTAB)
    nxz_flat = new_xyz.reshape(B * S, 3)

    outs = []
    for i, K in enumerate(NSAMPLE_LIST):
        m = B * S * K
        gi = jnp.transpose(gidx[i], (0, 2, 1)).reshape(-1)
        g = _sc_gather(table, gi, m)
        outs.append(_run_mlp(g, nxz_flat, params[i], K, m))

    new_points = jnp.concatenate(outs, axis=-1).reshape(B, S, -1)
    return (new_xyz, new_points)


# speedup vs baseline: 1.1305x; 1.0444x over previous
import functools

import jax
import jax.numpy as jnp
from jax import lax
from jax.experimental import pallas as pl
from jax.experimental.pallas import tpu as pltpu
from jax.experimental.pallas import tpu_sc as plsc

NPOINT = 512
RADIUS_LIST = [0.1, 0.2, 0.4]
NSAMPLE_LIST = [16, 32, 128]
B, N = 8, 2048
CPTS = 64
CTAB = 128
TM = 16384
GATHER_WIN = 128


def _fps_body(xt_ref, nx_ref):
    X = xt_ref[:, 0, :]
    Y = xt_ref[:, 1, :]
    Z = xt_ref[:, 2, :]
    n = X.shape[1]
    iota = lax.broadcasted_iota(jnp.int32, (1, n), 1)

    XYZ = jnp.concatenate([X, Y, Z], axis=0)

    def body(i, carry):
        dist, far = carry
        oh = iota == far
        oh3 = jnp.concatenate([oh, oh, oh], axis=0)
        csum = jnp.sum(jnp.where(oh3, XYZ, 0.0), axis=1, keepdims=True)
        cx = csum[0:B]
        cy = csum[B:2 * B]
        cz = csum[2 * B:]
        nx_ref[:, pl.ds(i, 1), :] = jnp.concatenate(
            [cx, cy, cz], axis=1)[:, None, :]
        dx = X - cx
        dy = Y - cy
        dz = Z - cz
        d = (dx * dx + dy * dy) + dz * dz
        dist = jnp.minimum(dist, d)
        maxv = jnp.max(dist, axis=1, keepdims=True)
        cand = jnp.where(dist == maxv, iota, n)
        far = jnp.min(cand, axis=1, keepdims=True)
        return dist, far

    dist0 = jnp.full((B, N), 1e10, dtype=jnp.float32)
    far0 = jnp.zeros((B, 1), dtype=jnp.int32)
    lax.fori_loop(0, NPOINT, body, (dist0, far0))


def _run_fps(xyz_t):
    return pl.pallas_call(
        _fps_body,
        out_shape=jax.ShapeDtypeStruct((B, NPOINT, 3), jnp.float32),
    )(xyz_t)


def _ballq_body(xt_ref, nx_ref, g16_ref, g32_ref, g128_ref, scr_ref):
    S = NPOINT
    P = xt_ref[0]
    C = nx_ref[0]
    b = pl.program_id(0)
    boff = b * N

    dotT = lax.dot_general(
        P.astype(jnp.bfloat16), C.astype(jnp.bfloat16),
        (((0,), (1,)), ((), ())), preferred_element_type=jnp.float32)
    c2 = (C[:, 0:1] * C[:, 0:1] + C[:, 1:2] * C[:, 1:2]) + C[:, 2:3] * C[:, 2:3]
    p2 = (P[0:1, :] * P[0:1, :] + P[1:2, :] * P[1:2, :]) + P[2:3, :] * P[2:3, :]
    sqr = ((-2.0 * dotT) + jnp.transpose(c2)) + jnp.transpose(p2)

    for r, K, gref in zip(RADIUS_LIST, NSAMPLE_LIST,
                          (g16_ref, g32_ref, g128_ref)):
        mask = (sqr <= jnp.float32(r * r)).astype(jnp.int32)
        c = mask
        sh = 1
        while sh < N:
            c = c + jnp.concatenate(
                [jnp.zeros((sh, S), jnp.int32), c[:-sh, :]], axis=0)
            sh *= 2

        cb = jnp.minimum(c, K + 1).astype(jnp.bfloat16)
        ones_row = jnp.ones((1, N), jnp.bfloat16)

        def kbody(j, _):
            base = j * 8
            for t in range(8):
                kb = (base + t).astype(jnp.bfloat16)
                cm = jnp.where(cb <= kb, jnp.bfloat16(1), jnp.bfloat16(0))
                cnt = lax.dot_general(
                    ones_row, cm, (((1,), (0,)), ((), ())),
                    preferred_element_type=jnp.float32).astype(jnp.int32)
                scr_ref[pl.ds(base + t, 1)] = cnt[None]
            return 0

        lax.fori_loop(0, K // 8, kbody, 0)
        idxs = scr_ref[0:K, 0, :]
        first = idxs[0:1, :]
        gref[0] = jnp.minimum(jnp.where(idxs == N, first, idxs), N - 1) + boff


def _run_ballq(xyz_t, new_xyz):
    S = NPOINT
    return pl.pallas_call(
        _ballq_body,
        grid=(B,),
        in_specs=[
            pl.BlockSpec((1, 3, N), lambda i: (i, 0, 0)),
            pl.BlockSpec((1, S, 3), lambda i: (i, 0, 0)),
        ],
        out_specs=[
            pl.BlockSpec((1, K, S), lambda i: (i, 0, 0)) for K in NSAMPLE_LIST
        ],
        out_shape=[
            jax.ShapeDtypeStruct((B, K, S), jnp.int32) for K in NSAMPLE_LIST
        ],
        scratch_shapes=[pltpu.VMEM((max(NSAMPLE_LIST), 1, S), jnp.int32)],
        compiler_params=pltpu.CompilerParams(
            dimension_semantics=("parallel",)),
    )(xyz_t, new_xyz)


def _sc_gather(table, gidx_flat, m):
    mesh = plsc.VectorSubcoreMesh(core_axis_name="core",
                                  subcore_axis_name="subcore")

    @functools.partial(
        pl.kernel, mesh=mesh,
        out_type=jax.ShapeDtypeStruct((m, CTAB), jnp.float32))
    def k(tab_hbm, idx_hbm, out_hbm):
        def body(i_vmem, o_vmem):
            pltpu.sync_copy(tab_hbm.at[i_vmem.at[0]], o_vmem)

        pltpu.emit_pipeline(
            body,
            grid=(m // GATHER_WIN,),
            in_specs=[pl.BlockSpec((1, GATHER_WIN), index_map=lambda i: (0, i))],
            out_specs=[pl.BlockSpec((GATHER_WIN, CTAB),
                                    index_map=lambda i: (i, 0))],
            core_axis_name=("core", "subcore"),
            dimension_semantics=(pltpu.PARALLEL,),
        )(idx_hbm, out_hbm)

    return k(table, gidx_flat.reshape(1, m))


def _layer1_body(g_ref, nxz_ref, w_ref, b_ref, x_ref, st_ref, *, K):
    g = g_ref[...]
    tk = TM // K
    cent = nxz_ref[...]
    centk = jnp.broadcast_to(cent[:, None, :], (tk, K, 3)).reshape(TM, 3)
    gxyz = g[:, 64:67] - centk
    xin = jnp.concatenate(
        [g[:, :64].astype(jnp.bfloat16), gxyz.astype(jnp.bfloat16),
         jnp.zeros((TM, CTAB - 67), jnp.bfloat16)], axis=1)
    x = lax.dot_general(
        xin, w_ref[...].astype(jnp.bfloat16),
        (((1,), (0,)), ((), ())), preferred_element_type=jnp.float32)
    x = x + b_ref[...]
    x_ref[...] = x.astype(jnp.bfloat16)
    c = x.shape[1]
    part = jnp.concatenate(
        [jnp.sum(x, axis=0, keepdims=True),
         jnp.sum(x * x, axis=0, keepdims=True),
         jnp.zeros((6, c), jnp.float32)], axis=0)

    @pl.when(pl.program_id(0) == 0)
    def _():
        st_ref[...] = part

    @pl.when(pl.program_id(0) != 0)
    def _():
        st_ref[...] = st_ref[...] + part


def _layer_mid_body(x_ref, st_in_ref, w_ref, b_ref, gamma_ref, beta_ref,
                    y_ref, st_ref, *, m_count):
    x = x_ref[...].astype(jnp.float32)
    s = st_in_ref[0:1, :]
    ss = st_in_ref[1:2, :]
    mean = s / m_count
    var = ss / m_count - mean * mean
    a = gamma_ref[...] / jnp.sqrt(var + 1e-5)
    cshift = beta_ref[...] - a * mean
    h = jnp.maximum(x * a + cshift, 0.0)
    y = lax.dot_general(
        h.astype(jnp.bfloat16), w_ref[...].astype(jnp.bfloat16),
        (((1,), (0,)), ((), ())), preferred_element_type=jnp.float32)
    y = y + b_ref[...]
    y_ref[...] = y.astype(jnp.bfloat16)
    c = y.shape[1]
    part = jnp.concatenate(
        [jnp.sum(y, axis=0, keepdims=True),
         jnp.sum(y * y, axis=0, keepdims=True),
         jnp.zeros((6, c), jnp.float32)], axis=0)

    @pl.when(pl.program_id(0) == 0)
    def _():
        st_ref[...] = part

    @pl.when(pl.program_id(0) != 0)
    def _():
        st_ref[...] = st_ref[...] + part


def _final_body(x_ref, st_in_ref, gamma_ref, beta_ref, o_ref, *, m_count, K):
    x = x_ref[...].astype(jnp.float32)
    s = st_in_ref[0:1, :]
    ss = st_in_ref[1:2, :]
    mean = s / m_count
    var = ss / m_count - mean * mean
    a = gamma_ref[...] / jnp.sqrt(var + 1e-5)
    cshift = beta_ref[...] - a * mean
    h = jnp.maximum(x * a + cshift, 0.0)
    c = x.shape[1]
    o_ref[...] = jnp.max(h.reshape(TM // K, K, c), axis=1)[None]


def _run_mlp(g, nxz_flat, params, K, m):
    steps = m // TM
    acc = pltpu.CompilerParams(dimension_semantics=("arbitrary",))
    full = lambda shape: pl.BlockSpec(shape, lambda i: tuple(0 for _ in shape))

    (w1, b1, g1, be1), (w2, b2, g2, be2), (w3, b3, g3, be3) = params
    c1, c2r, c3 = w1.shape[0], w2.shape[0], w3.shape[0]
    w1p = jnp.zeros((CTAB, c1), jnp.float32).at[:67, :].set(w1.T)
    w2p, w3p = w2.T, w3.T

    x1, st1 = pl.pallas_call(
        functools.partial(_layer1_body, K=K),
        grid=(steps,),
        in_specs=[
            pl.BlockSpec((TM, CTAB), lambda i: (i, 0)),
            pl.BlockSpec((TM // K, 3), lambda i: (i, 0)),
            full((CTAB, c1)),
            full((1, c1)),
        ],
        out_specs=[pl.BlockSpec((TM, c1), lambda i: (i, 0)), full((8, c1))],
        out_shape=[jax.ShapeDtypeStruct((m, c1), jnp.bfloat16),
                   jax.ShapeDtypeStruct((8, c1), jnp.float32)],
        compiler_params=acc,
    )(g, nxz_flat, w1p, b1[None, :])

    x2, st2 = pl.pallas_call(
        functools.partial(_layer_mid_body, m_count=float(m)),
        grid=(steps,),
        in_specs=[
            pl.BlockSpec((TM, c1), lambda i: (i, 0)),
            full((8, c1)),
            full((c1, c2r)),
            full((1, c2r)),
            full((1, c1)),
            full((1, c1)),
        ],
        out_specs=[pl.BlockSpec((TM, c2r), lambda i: (i, 0)), full((8, c2r))],
        out_shape=[jax.ShapeDtypeStruct((m, c2r), jnp.bfloat16),
                   jax.ShapeDtypeStruct((8, c2r), jnp.float32)],
        compiler_params=acc,
    )(x1, st1, w2p, b2[None, :], g1[None, :], be1[None, :])

    x3, st3 = pl.pallas_call(
        functools.partial(_layer_mid_body, m_count=float(m)),
        grid=(steps,),
        in_specs=[
            pl.BlockSpec((TM, c2r), lambda i: (i, 0)),
            full((8, c2r)),
            full((c2r, c3)),
            full((1, c3)),
            full((1, c2r)),
            full((1, c2r)),
        ],
        out_specs=[pl.BlockSpec((TM, c3), lambda i: (i, 0)), full((8, c3))],
        out_shape=[jax.ShapeDtypeStruct((m, c3), jnp.bfloat16),
                   jax.ShapeDtypeStruct((8, c3), jnp.float32)],
        compiler_params=acc,
    )(x2, st2, w3p, b3[None, :], g2[None, :], be2[None, :])

    out = pl.pallas_call(
        functools.partial(_final_body, m_count=float(m), K=K),
        grid=(steps,),
        in_specs=[
            pl.BlockSpec((TM, c3), lambda i: (i, 0)),
            full((8, c3)),
            full((1, c3)),
            full((1, c3)),
        ],
        out_specs=pl.BlockSpec((1, TM // K, c3), lambda i: (i, 0, 0)),
        out_shape=jax.ShapeDtypeStruct((steps, TM // K, c3), jnp.float32),
        compiler_params=acc,
    )(x3, st3, g3[None, :], be3[None, :])
    return out.reshape(B * NPOINT, c3)


def kernel(xyz, points, params):
    S = NPOINT
    xyz_t = jnp.transpose(xyz, (0, 2, 1))
    new_xyz = _run_fps(xyz_t)
    gidx = _run_ballq(xyz_t, new_xyz)

    table = jnp.concatenate(
        [points, xyz, jnp.zeros((B, N, CTAB - 67), jnp.float32)],
        axis=-1).reshape(B * N, CTAB)
    nxz_flat = new_xyz.reshape(B * S, 3)

    outs = []
    for i, K in enumerate(NSAMPLE_LIST):
        m = B * S * K
        gi = jnp.transpose(gidx[i], (0, 2, 1)).reshape(-1)
        g = _sc_gather(table, gi, m)
        outs.append(_run_mlp(g, nxz_flat, params[i], K, m))

    new_points = jnp.concatenate(outs, axis=-1).reshape(B, S, -1)
    return (new_xyz, new_points)
